# baseline (speedup 1.0000x reference)
import jax
import jax.numpy as jnp
from jax import lax
from jax.experimental import pallas as pl
from jax.experimental.pallas import tpu as pltpu
from jax.experimental.pallas import tpu_sc as plsc

N_NODES = 10000
N_PAD = 10240
N_EDGES = 320000
E_PAD = 32 * 80 * 128
BLK = 128
NBLK = 80
STRIPE = N_PAD // 16
GK = 8

_mesh = plsc.VectorSubcoreMesh(core_axis_name="c", subcore_axis_name="s")


def _deg_body(dst_hbm, zeros_hbm, ones_hbm, out_hbm, idx_v, ones_v, acc_sh, sem):
    cid = lax.axis_index("c")
    sid = lax.axis_index("s")
    wid = cid * 16 + sid
    pltpu.sync_copy(zeros_hbm, acc_sh.at[pl.ds(sid * STRIPE, STRIPE)])
    pltpu.sync_copy(ones_hbm, ones_v)
    pltpu.sync_copy(dst_hbm.at[wid], idx_v)
    plsc.subcore_barrier()

    @pl.loop(0, NBLK // GK)
    def _(gg):
        base = gg * GK
        hs = [pltpu.async_copy(ones_v, acc_sh.at[idx_v.at[base + k]], sem,
                               add=True)
              for k in range(GK)]
        for h in hs:
            h.wait()

    plsc.subcore_barrier()
    pltpu.sync_copy(acc_sh.at[pl.ds(sid * STRIPE, STRIPE)],
                    out_hbm.at[cid, pl.ds(sid * STRIPE, STRIPE)])


def _nrsqrt(d):
    i = lax.shift_right_logical(lax.bitcast_convert_type(d, jnp.int32), 1)
    y = lax.bitcast_convert_type(jnp.int32(0x5F3759DF) - i, jnp.float32)
    for _ in range(3):
        y = y * (1.5 - 0.5 * d * y * y)
    return y


def _msg1_body(src_hbm, dst_hbm, m1_hbm, zeros_hbm, zeros1_hbm, ones_hbm,
               out_hbm, dinv_hbm,
               idxs_v, idxd_v, idg0_v, idg1_v, ones_v, rows_v, m1_v, d0_v,
               deg_sh, table_sh, acc_sh, gsem, ssem):
    cid = lax.axis_index("c")
    sid = lax.axis_index("s")
    wid = cid * 16 + sid
    sl = pl.ds(sid * STRIPE, STRIPE)
    pltpu.sync_copy(zeros_hbm, acc_sh.at[sl])
    pltpu.sync_copy(zeros1_hbm, deg_sh.at[sl])
    pltpu.sync_copy(ones_hbm, ones_v)
    pltpu.sync_copy(m1_hbm.at[sl], m1_v)
    pltpu.sync_copy(dst_hbm.at[2 * sid], idg0_v)
    pltpu.sync_copy(dst_hbm.at[2 * sid + 1], idg1_v)
    pltpu.sync_copy(src_hbm.at[wid], idxs_v)
    pltpu.sync_copy(dst_hbm.at[wid], idxd_v)
    plsc.subcore_barrier()

    @pl.loop(0, NBLK // GK)
    def _(gg):
        base = gg * GK
        hs = [pltpu.async_copy(ones_v, deg_sh.at[idg0_v.at[base + k]], gsem,
                               add=True)
              for k in range(GK)]
        hs += [pltpu.async_copy(ones_v, deg_sh.at[idg1_v.at[base + k]], ssem,
                                add=True)
               for k in range(GK)]
        for h in hs:
            h.wait()

    plsc.subcore_barrier()
    pltpu.sync_copy(deg_sh.at[sl], d0_v)

    @pl.loop(0, STRIPE // 16)
    def _(i):
        s16 = pl.ds(i * 16, 16)
        dv16 = _nrsqrt(d0_v[s16] + 1.0)
        d0_v[s16] = dv16
        for r0 in range(16):
            r = i * 16 + r0
            m1_v[r] = m1_v[r] * dv16[r0]

    pltpu.sync_copy(m1_v, table_sh.at[sl])

    @pl.when(cid == 0)
    def _():
        pltpu.sync_copy(d0_v, dinv_hbm.at[sl])

    plsc.subcore_barrier()

    @pl.loop(0, NBLK // GK)
    def _(gg):
        base = gg * GK
        ghs = [pltpu.async_copy(table_sh.at[idxs_v.at[base + k]],
                                rows_v.at[pl.ds(k * BLK, BLK)], gsem)
               for k in range(GK)]
        for h in ghs:
            h.wait()
        shs = [pltpu.async_copy(rows_v.at[pl.ds(k * BLK, BLK)],
                                acc_sh.at[idxd_v.at[base + k]], ssem,
                                add=True)
               for k in range(GK)]
        for h in shs:
            h.wait()

    plsc.subcore_barrier()
    pltpu.sync_copy(acc_sh.at[sl], out_hbm.at[cid, sl])


def _sc_msg1(srcp, dstp, m1, zeros2, zeros1, ones128):
    return pl.kernel(
        _msg1_body,
        out_type=(jax.ShapeDtypeStruct((2, N_PAD, 16), jnp.float32),
                  jax.ShapeDtypeStruct((N_PAD,), jnp.float32)),
        mesh=_mesh,
        scratch_types=[
            pltpu.VMEM((NBLK, BLK), jnp.int32),
            pltpu.VMEM((NBLK, BLK), jnp.int32),
            pltpu.VMEM((NBLK, BLK), jnp.int32),
            pltpu.VMEM((NBLK, BLK), jnp.int32),
            pltpu.VMEM((BLK,), jnp.float32),
            pltpu.VMEM((GK * BLK, 16), jnp.float32),
            pltpu.VMEM((STRIPE, 16), jnp.float32),
            pltpu.VMEM((STRIPE,), jnp.float32),
            pltpu.VMEM_SHARED((N_PAD,), jnp.float32),
            pltpu.VMEM_SHARED((N_PAD, 16), jnp.float32),
            pltpu.VMEM_SHARED((N_PAD, 16), jnp.float32),
            pltpu.SemaphoreType.DMA,
            pltpu.SemaphoreType.DMA,
        ],
        compiler_params=pltpu.CompilerParams(use_tc_tiling_on_sc=False),
    )(srcp, dstp, m1, zeros2, zeros1, ones128)


def _msg_body(src_hbm, dst_hbm, table_hbm, zeros_hbm, out_hbm,
              idxs_v, idxd_v, rows_v, table_sh, acc_sh, gsem, ssem):
    cid = lax.axis_index("c")
    sid = lax.axis_index("s")
    wid = cid * 16 + sid
    pltpu.sync_copy(zeros_hbm, acc_sh.at[pl.ds(sid * STRIPE, STRIPE)])
    pltpu.sync_copy(table_hbm.at[pl.ds(sid * STRIPE, STRIPE)],
                    table_sh.at[pl.ds(sid * STRIPE, STRIPE)])
    pltpu.sync_copy(src_hbm.at[wid], idxs_v)
    pltpu.sync_copy(dst_hbm.at[wid], idxd_v)
    plsc.subcore_barrier()

    @pl.loop(0, NBLK // GK)
    def _(gg):
        base = gg * GK
        ghs = [pltpu.async_copy(table_sh.at[idxs_v.at[base + k]],
                                rows_v.at[pl.ds(k * BLK, BLK)], gsem)
               for k in range(GK)]
        for h in ghs:
            h.wait()
        shs = [pltpu.async_copy(rows_v.at[pl.ds(k * BLK, BLK)],
                                acc_sh.at[idxd_v.at[base + k]], ssem,
                                add=True)
               for k in range(GK)]
        for h in shs:
            h.wait()

    plsc.subcore_barrier()
    pltpu.sync_copy(acc_sh.at[pl.ds(sid * STRIPE, STRIPE)],
                    out_hbm.at[cid, pl.ds(sid * STRIPE, STRIPE)])


def _sc_degree(dstp, zeros1, ones128):
    return pl.kernel(
        _deg_body,
        out_type=jax.ShapeDtypeStruct((2, N_PAD), jnp.float32),
        mesh=_mesh,
        scratch_types=[
            pltpu.VMEM((NBLK, BLK), jnp.int32),
            pltpu.VMEM((BLK,), jnp.float32),
            pltpu.VMEM_SHARED((N_PAD,), jnp.float32),
            pltpu.SemaphoreType.DMA,
        ],
    )(dstp, zeros1, ones128)


def _sc_message(srcp, dstp, table, zeros2):
    return pl.kernel(
        _msg_body,
        out_type=jax.ShapeDtypeStruct((2, N_PAD, 16), jnp.float32),
        mesh=_mesh,
        scratch_types=[
            pltpu.VMEM((NBLK, BLK), jnp.int32),
            pltpu.VMEM((NBLK, BLK), jnp.int32),
            pltpu.VMEM((GK * BLK, 16), jnp.float32),
            pltpu.VMEM_SHARED((N_PAD, 16), jnp.float32),
            pltpu.VMEM_SHARED((N_PAD, 16), jnp.float32),
            pltpu.SemaphoreType.DMA,
            pltpu.SemaphoreType.DMA,
        ],
        compiler_params=pltpu.CompilerParams(use_tc_tiling_on_sc=False),
    )(srcp, dstp, table, zeros2)


def _mm1_body(x_ref, w_ref, o_ref):
    o_ref[...] = jnp.dot(x_ref[...], w_ref[...],
                         preferred_element_type=jnp.float32)


def _tc_mm1(xp, W1):
    return pl.pallas_call(
        _mm1_body,
        grid=(8,),
        in_specs=[
            pl.BlockSpec((N_PAD // 8, 128), lambda i: (i, 0)),
            pl.BlockSpec((128, 16), lambda i: (0, 0)),
        ],
        out_specs=pl.BlockSpec((N_PAD // 8, 16), lambda i: (i, 0)),
        out_shape=jax.ShapeDtypeStruct((N_PAD, 16), jnp.float32),
    )(xp, W1)


def _layer2_body(dv_ref, m_ref, p_ref, b1_ref, w2_ref, o_ref):
    dv = jnp.broadcast_to(dv_ref[...], (N_PAD, 16))
    g1 = m_ref[...] * dv
    s = p_ref[0] + p_ref[1] + g1
    t = jnp.maximum(s * dv + b1_ref[...], 0.0)
    m2 = jnp.dot(t, w2_ref[...], preferred_element_type=jnp.float32)
    g2 = m2 * dv[:, :8]
    o_ref[...] = jnp.concatenate(
        [g2, jnp.zeros((N_PAD, 8), jnp.float32)], axis=1)


def _tc_layer2(dinv3, m1, p, b1r, W2):
    return pl.pallas_call(
        _layer2_body,
        out_shape=jax.ShapeDtypeStruct((N_PAD, 16), jnp.float32),
    )(dinv3, m1, p, b1r, W2)


def _final_body(dv_ref, q_ref, g2_ref, b2_ref, wfc_ref, bfc_ref, o_ref):
    dv = jnp.broadcast_to(dv_ref[...], (N_PAD, 8))
    s = q_ref[0, :, :8] + q_ref[1, :, :8] + g2_ref[:, :8]
    h2 = jnp.maximum(s * dv + b2_ref[...], 0.0)
    logits = jnp.dot(h2, wfc_ref[...],
                     preferred_element_type=jnp.float32) + bfc_ref[...]
    m = jnp.max(logits, axis=1, keepdims=True)
    e = jnp.exp(logits - m)
    lse = jnp.log(jnp.sum(e, axis=1, keepdims=True)) + m
    o_ref[...] = logits - lse


def _tc_final(dinv3, q, g2p, b2r, Wfc, bfcr):
    return pl.pallas_call(
        _final_body,
        out_shape=jax.ShapeDtypeStruct((N_PAD, 2), jnp.float32),
    )(dinv3, q, g2p, b2r, Wfc, bfcr)


def kernel(x, edge_index, W1, b1, W2, b2, Wfc, bfc):
    ei = edge_index.astype(jnp.int32)
    npad = E_PAD - N_EDGES
    fill = jnp.arange(npad, dtype=jnp.int32) % 16
    srcp = jnp.concatenate([ei[0], fill]).reshape(32, NBLK, BLK)
    dstp = jnp.concatenate([ei[1], N_NODES + fill]).reshape(32, NBLK, BLK)

    xp = jnp.pad(x, ((0, N_PAD - N_NODES), (0, 0)))
    zeros1 = jnp.zeros((STRIPE,), jnp.float32)
    zeros2 = jnp.zeros((STRIPE, 16), jnp.float32)
    ones128 = jnp.ones((BLK,), jnp.float32)

    m1 = _tc_mm1(xp, W1)

    p, dinv = _sc_msg1(srcp, dstp, m1, zeros2, zeros1, ones128)
    dinv3 = dinv.reshape(N_PAD, 1)
    g2p = _tc_layer2(dinv3, m1, p, b1.reshape(1, 16), W2)

    q = _sc_message(srcp, dstp, g2p, zeros2)
    out = _tc_final(dinv3, q, g2p, b2.reshape(1, 8), Wfc, bfc.reshape(1, 2))
    return out[:N_NODES]


# speedup vs baseline: 1.0374x; 1.0374x over previous
import jax
import jax.numpy as jnp
from jax import lax
from jax.experimental import pallas as pl
from jax.experimental.pallas import tpu as pltpu
from jax.experimental.pallas import tpu_sc as plsc

N_NODES = 10000
N_PAD = 10240
N_EDGES = 320000
E_PAD = 32 * 80 * 128
BLK = 128
NBLK = 80
STRIPE = N_PAD // 16
GK = 8

def _mesh():
    return plsc.VectorSubcoreMesh(core_axis_name="c", subcore_axis_name="s")


def _deg_body(dst_hbm, zeros_hbm, ones_hbm, out_hbm, idx_v, ones_v, acc_sh, sem):
    cid = lax.axis_index("c")
    sid = lax.axis_index("s")
    wid = cid * 16 + sid
    pltpu.sync_copy(zeros_hbm, acc_sh.at[pl.ds(sid * STRIPE, STRIPE)])
    pltpu.sync_copy(ones_hbm, ones_v)
    pltpu.sync_copy(dst_hbm.at[wid], idx_v)
    plsc.subcore_barrier()

    @pl.loop(0, NBLK // GK)
    def _(gg):
        base = gg * GK
        hs = [pltpu.async_copy(ones_v, acc_sh.at[idx_v.at[base + k]], sem,
                               add=True)
              for k in range(GK)]
        for h in hs:
            h.wait()

    plsc.subcore_barrier()
    pltpu.sync_copy(acc_sh.at[pl.ds(sid * STRIPE, STRIPE)],
                    out_hbm.at[cid, pl.ds(sid * STRIPE, STRIPE)])


def _nrsqrt(d):
    i = lax.shift_right_logical(lax.bitcast_convert_type(d, jnp.int32), 1)
    y = lax.bitcast_convert_type(jnp.int32(0x5F3759DF) - i, jnp.float32)
    for _ in range(3):
        y = y * (1.5 - 0.5 * d * y * y)
    return y


def _msg1_body(src_hbm, dst_hbm, m1_hbm, dp_hbm, zeros_hbm, out_hbm,
               idxs_v, idxd_v, rows_v, m1_v, d0_v, d1_v, table_sh, acc_sh,
               gsem, ssem):
    cid = lax.axis_index("c")
    sid = lax.axis_index("s")
    wid = cid * 16 + sid
    sl = pl.ds(sid * STRIPE, STRIPE)
    pltpu.sync_copy(zeros_hbm, acc_sh.at[sl])
    pltpu.sync_copy(m1_hbm.at[sl], m1_v)
    pltpu.sync_copy(dp_hbm.at[0, sl], d0_v)
    pltpu.sync_copy(dp_hbm.at[1, sl], d1_v)
    pltpu.sync_copy(src_hbm.at[wid], idxs_v)
    pltpu.sync_copy(dst_hbm.at[wid], idxd_v)

    @pl.loop(0, STRIPE // 16)
    def _(i):
        s16 = pl.ds(i * 16, 16)
        dv16 = _nrsqrt(d0_v[s16] + d1_v[s16] + 1.0)
        for r0 in range(16):
            r = i * 16 + r0
            m1_v[r] = m1_v[r] * dv16[r0]

    pltpu.sync_copy(m1_v, table_sh.at[sl])
    plsc.subcore_barrier()

    @pl.loop(0, NBLK // GK)
    def _(gg):
        base = gg * GK
        ghs = [pltpu.async_copy(table_sh.at[idxs_v.at[base + k]],
                                rows_v.at[pl.ds(k * BLK, BLK)], gsem)
               for k in range(GK)]
        for h in ghs:
            h.wait()
        shs = [pltpu.async_copy(rows_v.at[pl.ds(k * BLK, BLK)],
                                acc_sh.at[idxd_v.at[base + k]], ssem,
                                add=True)
               for k in range(GK)]
        for h in shs:
            h.wait()

    plsc.subcore_barrier()
    pltpu.sync_copy(acc_sh.at[sl], out_hbm.at[cid, sl])


def _sc_msg1(srcp, dstp, m1, dpart, zeros2):
    return pl.kernel(
        _msg1_body,
        out_type=jax.ShapeDtypeStruct((2, N_PAD, 16), jnp.float32),
        mesh=_mesh(),
        scratch_types=[
            pltpu.VMEM((NBLK, BLK), jnp.int32),
            pltpu.VMEM((NBLK, BLK), jnp.int32),
            pltpu.VMEM((GK * BLK, 16), jnp.float32),
            pltpu.VMEM((STRIPE, 16), jnp.float32),
            pltpu.VMEM((STRIPE,), jnp.float32),
            pltpu.VMEM((STRIPE,), jnp.float32),
            pltpu.VMEM_SHARED((N_PAD, 16), jnp.float32),
            pltpu.VMEM_SHARED((N_PAD, 16), jnp.float32),
            pltpu.SemaphoreType.DMA,
            pltpu.SemaphoreType.DMA,
        ],
        compiler_params=pltpu.CompilerParams(use_tc_tiling_on_sc=False),
    )(srcp, dstp, m1, dpart, zeros2)


def _msg_body(src_hbm, dst_hbm, table_hbm, zeros_hbm, out_hbm,
              idxs_v, idxd_v, rows_v, table_sh, acc_sh, gsem, ssem):
    cid = lax.axis_index("c")
    sid = lax.axis_index("s")
    wid = cid * 16 + sid
    pltpu.sync_copy(zeros_hbm, acc_sh.at[pl.ds(sid * STRIPE, STRIPE)])
    pltpu.sync_copy(table_hbm.at[pl.ds(sid * STRIPE, STRIPE)],
                    table_sh.at[pl.ds(sid * STRIPE, STRIPE)])
    pltpu.sync_copy(src_hbm.at[wid], idxs_v)
    pltpu.sync_copy(dst_hbm.at[wid], idxd_v)
    plsc.subcore_barrier()

    @pl.loop(0, NBLK // GK)
    def _(gg):
        base = gg * GK
        ghs = [pltpu.async_copy(table_sh.at[idxs_v.at[base + k]],
                                rows_v.at[pl.ds(k * BLK, BLK)], gsem)
               for k in range(GK)]
        for h in ghs:
            h.wait()
        shs = [pltpu.async_copy(rows_v.at[pl.ds(k * BLK, BLK)],
                                acc_sh.at[idxd_v.at[base + k]], ssem,
                                add=True)
               for k in range(GK)]
        for h in shs:
            h.wait()

    plsc.subcore_barrier()
    pltpu.sync_copy(acc_sh.at[pl.ds(sid * STRIPE, STRIPE)],
                    out_hbm.at[cid, pl.ds(sid * STRIPE, STRIPE)])


def _sc_degree(dstp, zeros1, ones128):
    return pl.kernel(
        _deg_body,
        out_type=jax.ShapeDtypeStruct((2, N_PAD), jnp.float32),
        mesh=_mesh(),
        scratch_types=[
            pltpu.VMEM((NBLK, BLK), jnp.int32),
            pltpu.VMEM((BLK,), jnp.float32),
            pltpu.VMEM_SHARED((N_PAD,), jnp.float32),
            pltpu.SemaphoreType.DMA,
        ],
    )(dstp, zeros1, ones128)


def _sc_message(srcp, dstp, table, zeros2):
    return pl.kernel(
        _msg_body,
        out_type=jax.ShapeDtypeStruct((2, N_PAD, 16), jnp.float32),
        mesh=_mesh(),
        scratch_types=[
            pltpu.VMEM((NBLK, BLK), jnp.int32),
            pltpu.VMEM((NBLK, BLK), jnp.int32),
            pltpu.VMEM((GK * BLK, 16), jnp.float32),
            pltpu.VMEM_SHARED((N_PAD, 16), jnp.float32),
            pltpu.VMEM_SHARED((N_PAD, 16), jnp.float32),
            pltpu.SemaphoreType.DMA,
            pltpu.SemaphoreType.DMA,
        ],
        compiler_params=pltpu.CompilerParams(use_tc_tiling_on_sc=False),
    )(srcp, dstp, table, zeros2)


def _mm1_body(x_ref, w_ref, o_ref):
    o_ref[...] = jnp.dot(x_ref[...], w_ref[...],
                         preferred_element_type=jnp.float32)


def _tc_mm1(xp, W1):
    return pl.pallas_call(
        _mm1_body,
        grid=(8,),
        in_specs=[
            pl.BlockSpec((N_PAD // 8, 128), lambda i: (i, 0)),
            pl.BlockSpec((128, 16), lambda i: (0, 0)),
        ],
        out_specs=pl.BlockSpec((N_PAD // 8, 16), lambda i: (i, 0)),
        out_shape=jax.ShapeDtypeStruct((N_PAD, 16), jnp.float32),
    )(xp, W1)


def _layer2_body(dp_ref, m_ref, p_ref, b1_ref, w2_ref, o_ref):
    dinv = lax.rsqrt(dp_ref[0] + dp_ref[1] + 1.0)
    dv = jnp.broadcast_to(dinv, (N_PAD, 16))
    g1 = m_ref[...] * dv
    s = p_ref[0] + p_ref[1] + g1
    t = jnp.maximum(s * dv + b1_ref[...], 0.0)
    m2 = jnp.dot(t, w2_ref[...], preferred_element_type=jnp.float32)
    g2 = m2 * dv[:, :8]
    o_ref[...] = jnp.concatenate(
        [g2, jnp.zeros((N_PAD, 8), jnp.float32)], axis=1)


def _tc_layer2(dpart3, m1, p, b1r, W2):
    return pl.pallas_call(
        _layer2_body,
        out_shape=jax.ShapeDtypeStruct((N_PAD, 16), jnp.float32),
    )(dpart3, m1, p, b1r, W2)


def _final_body(dp_ref, q_ref, g2_ref, b2_ref, wfc_ref, bfc_ref, o_ref):
    dinv = lax.rsqrt(dp_ref[0] + dp_ref[1] + 1.0)
    dv = jnp.broadcast_to(dinv, (N_PAD, 8))
    s = q_ref[0, :, :8] + q_ref[1, :, :8] + g2_ref[:, :8]
    h2 = jnp.maximum(s * dv + b2_ref[...], 0.0)
    logits = jnp.dot(h2, wfc_ref[...],
                     preferred_element_type=jnp.float32) + bfc_ref[...]
    m = jnp.max(logits, axis=1, keepdims=True)
    e = jnp.exp(logits - m)
    lse = jnp.log(jnp.sum(e, axis=1, keepdims=True)) + m
    o_ref[...] = logits - lse


def _tc_final(dpart3, q, g2p, b2r, Wfc, bfcr):
    return pl.pallas_call(
        _final_body,
        out_shape=jax.ShapeDtypeStruct((N_PAD, 2), jnp.float32),
    )(dpart3, q, g2p, b2r, Wfc, bfcr)


def kernel(x, edge_index, W1, b1, W2, b2, Wfc, bfc):
    ei = edge_index.astype(jnp.int32)
    npad = E_PAD - N_EDGES
    fill = jnp.arange(npad, dtype=jnp.int32) % 16
    srcp = jnp.concatenate([ei[0], fill]).reshape(32, NBLK, BLK)
    dstp = jnp.concatenate([ei[1], N_NODES + fill]).reshape(32, NBLK, BLK)

    xp = jnp.pad(x, ((0, N_PAD - N_NODES), (0, 0)))
    zeros1 = jnp.zeros((STRIPE,), jnp.float32)
    zeros2 = jnp.zeros((STRIPE, 16), jnp.float32)
    ones128 = jnp.ones((BLK,), jnp.float32)

    m1 = _tc_mm1(xp, W1)
    dpart = _sc_degree(dstp, zeros1, ones128)
    dpart3 = dpart.reshape(2, N_PAD, 1)

    p = _sc_msg1(srcp, dstp, m1, dpart, zeros2)
    g2p = _tc_layer2(dpart3, m1, p, b1.reshape(1, 16), W2)

    q = _sc_message(srcp, dstp, g2p, zeros2)
    out = _tc_final(dpart3, q, g2p, b2.reshape(1, 8), Wfc, bfc.reshape(1, 2))
    return out[:N_NODES]
